# baseline (speedup 1.0000x reference)
import functools
import math

import jax
import jax.numpy as jnp
from jax import lax
from jax.experimental import pallas as pl
from jax.experimental.pallas import tpu as pltpu
from jax.experimental.pallas import tpu_sc as plsc

_L = 16
_NW = 32
_C = 4096

_C00 = 0.5 * math.sqrt(1.0 / math.pi)
_C1 = math.sqrt(3.0 / (4.0 * math.pi))
_C2A = 0.5 * math.sqrt(15.0 / math.pi)
_C20 = 0.25 * math.sqrt(5.0 / math.pi)
_C22 = 0.25 * math.sqrt(15.0 / math.pi)


def _rsqrt(r2):
    i = plsc.bitcast(r2, jnp.int32)
    i = jnp.int32(0x5F3759DF) - lax.shift_right_logical(i, 1)
    y = plsc.bitcast(i, jnp.float32)
    h = 0.5 * r2
    y = y * (1.5 - h * y * y)
    y = y * (1.5 - h * y * y)
    return y


@functools.lru_cache(maxsize=None)
def _make_sph(npts):
    p_per_w = npts // _NW
    chunk = min(_C, p_per_w)
    nch = p_per_w // chunk
    mesh = plsc.VectorSubcoreMesh(core_axis_name="c", subcore_axis_name="s")

    @functools.partial(
        pl.kernel,
        out_type=jax.ShapeDtypeStruct((npts * 9,), jnp.float32),
        mesh=mesh,
        scratch_types=[
            pltpu.VMEM((chunk * 3,), jnp.float32),
            pltpu.VMEM((chunk * 9,), jnp.float32),
        ],
        compiler_params=pltpu.CompilerParams(needs_layout_passes=False),
    )
    def sph(x_hbm, out_hbm, in_v, out_v):
        wid = lax.axis_index("s") * mesh.num_cores + lax.axis_index("c")
        lane = lax.iota(jnp.int32, _L)
        gidx0 = lane * 3
        sidx0 = lane * 9
        y0 = jnp.full((_L,), _C00, jnp.float32)

        def do_chunk(j, carry):
            base = wid * p_per_w + j * chunk
            pltpu.sync_copy(x_hbm.at[pl.ds(base * 3, chunk * 3)], in_v)

            def vec(v, c):
                gi = gidx0 + v * (_L * 3)
                x = plsc.load_gather(in_v, [gi])
                y = plsc.load_gather(in_v, [gi + 1])
                z = plsc.load_gather(in_v, [gi + 2])
                r2 = x * x + y * y + z * z + 1e-12
                rin = _rsqrt(r2)
                xn = x * rin
                yn = y * rin
                zn = z * rin
                si = sidx0 + v * (_L * 9)
                plsc.store_scatter(out_v, [si], y0)
                plsc.store_scatter(out_v, [si + 1], _C1 * yn)
                plsc.store_scatter(out_v, [si + 2], _C1 * zn)
                plsc.store_scatter(out_v, [si + 3], _C1 * xn)
                plsc.store_scatter(out_v, [si + 4], _C2A * (xn * yn))
                plsc.store_scatter(out_v, [si + 5], _C2A * (yn * zn))
                plsc.store_scatter(out_v, [si + 6], _C20 * (3.0 * (zn * zn) - 1.0))
                plsc.store_scatter(out_v, [si + 7], _C2A * (xn * zn))
                plsc.store_scatter(out_v, [si + 8], _C22 * (xn * xn - yn * yn))
                return c

            lax.fori_loop(0, chunk // _L, vec, 0)
            pltpu.sync_copy(out_v, out_hbm.at[pl.ds(base * 9, chunk * 9)])
            return carry

        lax.fori_loop(0, nch, do_chunk, 0)

    return sph


def kernel(X):
    b, n1, n2, _ = X.shape
    npts = b * n1 * n2
    xf = X.reshape(npts * 3)
    out = _make_sph(npts)(xf)
    return out.reshape(b, n1, n2, 9)


# speedup vs baseline: 19.4430x; 19.4430x over previous
import functools
import math

import jax
import jax.numpy as jnp
from jax import lax
from jax.experimental import pallas as pl
from jax.experimental.pallas import tpu as pltpu
from jax.experimental.pallas import tpu_sc as plsc

_L = 16
_NW = 32
_C = 4096

_C00 = 0.5 * math.sqrt(1.0 / math.pi)
_C1 = math.sqrt(3.0 / (4.0 * math.pi))
_C2A = 0.5 * math.sqrt(15.0 / math.pi)
_C20 = 0.25 * math.sqrt(5.0 / math.pi)
_C22 = 0.25 * math.sqrt(15.0 / math.pi)


def _rsqrt(r2):
    i = plsc.bitcast(r2, jnp.int32)
    i = jnp.int32(0x5F3759DF) - lax.shift_right_logical(i, 1)
    y = plsc.bitcast(i, jnp.float32)
    h = 0.5 * r2
    y = y * (1.5 - h * y * y)
    y = y * (1.5 - h * y * y)
    return y


@functools.lru_cache(maxsize=None)
def _make_sph(nb, plane):
    npts = nb * plane
    span = npts // _NW
    w_per_b = _NW // nb
    assert span * _NW == npts and w_per_b * nb == _NW
    chunk = min(_C, span)
    nch = span // chunk
    mesh = plsc.VectorSubcoreMesh(core_axis_name="c", subcore_axis_name="s")

    @functools.partial(
        pl.kernel,
        out_type=jax.ShapeDtypeStruct((npts * 9,), jnp.float32),
        mesh=mesh,
        scratch_types=[
            pltpu.VMEM((chunk,), jnp.float32),
            pltpu.VMEM((chunk,), jnp.float32),
            pltpu.VMEM((chunk,), jnp.float32),
            pltpu.VMEM((9 * chunk,), jnp.float32),
        ],
        compiler_params=pltpu.CompilerParams(needs_layout_passes=False),
    )
    def sph(x_hbm, out_hbm, xv, yv, zv, ov):
        wid = lax.axis_index("s") * mesh.num_cores + lax.axis_index("c")
        b = wid // w_per_b
        e0 = (wid % w_per_b) * span

        def do_chunk(j, carry):
            e = e0 + j * chunk
            pltpu.sync_copy(x_hbm.at[pl.ds((b * 3 + 0) * plane + e, chunk)], xv)
            pltpu.sync_copy(x_hbm.at[pl.ds((b * 3 + 1) * plane + e, chunk)], yv)
            pltpu.sync_copy(x_hbm.at[pl.ds((b * 3 + 2) * plane + e, chunk)], zv)

            def vec(v, c):
                s = v * _L
                x = xv[pl.ds(s, _L)]
                y = yv[pl.ds(s, _L)]
                z = zv[pl.ds(s, _L)]
                r2 = x * x + y * y + z * z + 1e-12
                rin = _rsqrt(r2)
                xn = x * rin
                yn = y * rin
                zn = z * rin
                ov[pl.ds(s, _L)] = jnp.full((_L,), _C00, jnp.float32)
                ov[pl.ds(chunk + s, _L)] = _C1 * yn
                ov[pl.ds(2 * chunk + s, _L)] = _C1 * zn
                ov[pl.ds(3 * chunk + s, _L)] = _C1 * xn
                ov[pl.ds(4 * chunk + s, _L)] = _C2A * (xn * yn)
                ov[pl.ds(5 * chunk + s, _L)] = _C2A * (yn * zn)
                ov[pl.ds(6 * chunk + s, _L)] = _C20 * (3.0 * (zn * zn) - 1.0)
                ov[pl.ds(7 * chunk + s, _L)] = _C2A * (xn * zn)
                ov[pl.ds(8 * chunk + s, _L)] = _C22 * (xn * xn - yn * yn)
                return c

            lax.fori_loop(0, chunk // _L, vec, 0)
            for k in range(9):
                pltpu.sync_copy(
                    ov.at[pl.ds(k * chunk, chunk)],
                    out_hbm.at[pl.ds((b * 9 + k) * plane + e, chunk)],
                )
            return carry

        lax.fori_loop(0, nch, do_chunk, 0)

    return sph


def kernel(X):
    b, n1, n2, _ = X.shape
    plane = n1 * n2
    xp = jnp.transpose(X, (0, 3, 1, 2))
    xp = xp.reshape(b, 3, n1 // 8, 8, n2 // 128, 128)
    xp = jnp.transpose(xp, (0, 1, 2, 4, 3, 5))
    xf = xp.reshape(b * plane * 3)
    of = _make_sph(b, plane)(xf)
    o = of.reshape(b, 9, n1 // 8, n2 // 128, 8, 128)
    o = jnp.transpose(o, (0, 2, 4, 3, 5, 1))
    return o.reshape(b, n1, n2, 9)


# speedup vs baseline: 26.3711x; 1.3563x over previous
import functools
import math

import jax
import jax.numpy as jnp
from jax import lax
from jax.experimental import pallas as pl
from jax.experimental.pallas import tpu as pltpu
from jax.experimental.pallas import tpu_sc as plsc

_L = 16
_NW = 32
_C = 4096

_C00 = 0.5 * math.sqrt(1.0 / math.pi)
_C1 = math.sqrt(3.0 / (4.0 * math.pi))
_C2A = 0.5 * math.sqrt(15.0 / math.pi)
_C20 = 0.25 * math.sqrt(5.0 / math.pi)
_C22 = 0.25 * math.sqrt(15.0 / math.pi)


def _rsqrt(r2):
    i = plsc.bitcast(r2, jnp.int32)
    i = jnp.int32(0x5F3759DF) - lax.shift_right_logical(i, 1)
    y = plsc.bitcast(i, jnp.float32)
    h = 0.5 * r2
    y = y * (1.5 - h * y * y)
    y = y * (1.5 - h * y * y)
    return y


@functools.lru_cache(maxsize=None)
def _make_sph(nb, plane):
    npts = nb * plane
    span = npts // _NW
    w_per_b = _NW // nb
    assert span * _NW == npts and w_per_b * nb == _NW
    chunk = min(_C, span)
    nch = span // chunk
    mesh = plsc.VectorSubcoreMesh(core_axis_name="c", subcore_axis_name="s")

    @functools.partial(
        pl.kernel,
        out_type=jax.ShapeDtypeStruct((npts * 9,), jnp.float32),
        mesh=mesh,
        scratch_types=[
            pltpu.VMEM((3 * chunk,), jnp.float32),
            pltpu.VMEM((3 * chunk,), jnp.float32),
            pltpu.VMEM((9 * chunk,), jnp.float32),
            pltpu.VMEM((9 * chunk,), jnp.float32),
            pltpu.SemaphoreType.DMA,
            pltpu.SemaphoreType.DMA,
            pltpu.SemaphoreType.DMA,
            pltpu.SemaphoreType.DMA,
        ],
        compiler_params=pltpu.CompilerParams(needs_layout_passes=False),
    )
    def sph(x_hbm, out_hbm, in0, in1, ov0, ov1, si0, si1, so0, so1):
        wid = lax.axis_index("s") * mesh.num_cores + lax.axis_index("c")
        b = wid // w_per_b
        e0 = (wid % w_per_b) * span
        ins = [in0, in1]
        ovs = [ov0, ov1]
        sis = [si0, si1]
        sos = [so0, so1]

        y0 = jnp.full((_L,), _C00, jnp.float32)

        def fill0(v, c):
            ov0[pl.ds(v * _L, _L)] = y0
            ov1[pl.ds(v * _L, _L)] = y0
            return c

        lax.fori_loop(0, chunk // _L, fill0, 0)

        def start_in(j):
            e = e0 + j * chunk
            p = j % 2
            return [
                pltpu.async_copy(
                    x_hbm.at[pl.ds((b * 3 + k) * plane + e, chunk)],
                    ins[p].at[pl.ds(k * chunk, chunk)],
                    sis[p],
                )
                for k in range(3)
            ]

        def start_out(j):
            e = e0 + j * chunk
            p = j % 2
            return [
                pltpu.async_copy(
                    ovs[p].at[pl.ds(k * chunk, chunk)],
                    out_hbm.at[pl.ds((b * 9 + k) * plane + e, chunk)],
                    sos[p],
                )
                for k in range(9)
            ]

        def compute(p):
            iv = ins[p]
            ov = ovs[p]

            def vec(v, c):
                s = v * _L
                x = iv[pl.ds(s, _L)]
                y = iv[pl.ds(chunk + s, _L)]
                z = iv[pl.ds(2 * chunk + s, _L)]
                r2 = x * x + y * y + z * z + 1e-12
                rin = _rsqrt(r2)
                xn = x * rin
                yn = y * rin
                zn = z * rin
                ov[pl.ds(chunk + s, _L)] = _C1 * yn
                ov[pl.ds(2 * chunk + s, _L)] = _C1 * zn
                ov[pl.ds(3 * chunk + s, _L)] = _C1 * xn
                ov[pl.ds(4 * chunk + s, _L)] = _C2A * (xn * yn)
                ov[pl.ds(5 * chunk + s, _L)] = _C2A * (yn * zn)
                ov[pl.ds(6 * chunk + s, _L)] = _C20 * (3.0 * (zn * zn) - 1.0)
                ov[pl.ds(7 * chunk + s, _L)] = _C2A * (xn * zn)
                ov[pl.ds(8 * chunk + s, _L)] = _C22 * (xn * xn - yn * yn)
                return c

            lax.fori_loop(0, chunk // _L, vec, 0)

        hin = {0: start_in(0)}
        hout = {}
        for j in range(nch):
            p = j % 2
            for h in hin.pop(j):
                h.wait()
            if j + 1 < nch:
                hin[j + 1] = start_in(j + 1)
            if j >= 2:
                for h in hout.pop(j - 2):
                    h.wait()
            compute(p)
            hout[j] = start_out(j)
        for j in sorted(hout):
            for h in hout.pop(j):
                h.wait()

    return sph


def kernel(X):
    b, n1, n2, _ = X.shape
    plane = n1 * n2
    xp = jnp.transpose(X, (0, 3, 1, 2))
    xp = xp.reshape(b, 3, n1 // 8, 8, n2 // 128, 128)
    xp = jnp.transpose(xp, (0, 1, 2, 4, 3, 5))
    xf = xp.reshape(b * plane * 3)
    of = _make_sph(b, plane)(xf)
    o = of.reshape(b, 9, n1 // 8, n2 // 128, 8, 128)
    o = jnp.transpose(o, (0, 2, 4, 3, 5, 1))
    return o.reshape(b, n1, n2, 9)


# speedup vs baseline: 47.0890x; 1.7856x over previous
import functools
import math

import jax
import jax.numpy as jnp
from jax import lax
from jax.experimental import pallas as pl
from jax.experimental.pallas import tpu as pltpu
from jax.experimental.pallas import tpu_sc as plsc

_L = 16
_NW = 32
_C = 4096

_C00 = 0.5 * math.sqrt(1.0 / math.pi)
_C1 = math.sqrt(3.0 / (4.0 * math.pi))
_C2A = 0.5 * math.sqrt(15.0 / math.pi)
_C20 = 0.25 * math.sqrt(5.0 / math.pi)
_C22 = 0.25 * math.sqrt(15.0 / math.pi)


def _rsqrt(r2):
    i = plsc.bitcast(r2, jnp.int32)
    i = jnp.int32(0x5F3759DF) - lax.shift_right_logical(i, 1)
    y = plsc.bitcast(i, jnp.float32)
    h = 0.5 * r2
    y = y * (1.5 - h * y * y)
    y = y * (1.5 - h * y * y)
    return y


@functools.lru_cache(maxsize=None)
def _make_sph(nb, plane):
    npts = nb * plane
    span = npts // _NW
    w_per_b = _NW // nb
    assert span * _NW == npts and w_per_b * nb == _NW
    chunk = min(_C, span)
    nch = span // chunk
    mesh = plsc.VectorSubcoreMesh(core_axis_name="c", subcore_axis_name="s")

    @functools.partial(
        pl.kernel,
        out_type=jax.ShapeDtypeStruct((npts * 9,), jnp.float32),
        mesh=mesh,
        scratch_types=[
            pltpu.VMEM((3 * chunk,), jnp.float32),
            pltpu.VMEM((3 * chunk,), jnp.float32),
            pltpu.VMEM((9 * chunk,), jnp.float32),
            pltpu.VMEM((9 * chunk,), jnp.float32),
            pltpu.SemaphoreType.DMA,
            pltpu.SemaphoreType.DMA,
            pltpu.SemaphoreType.DMA,
            pltpu.SemaphoreType.DMA,
        ],
        compiler_params=pltpu.CompilerParams(needs_layout_passes=False),
    )
    def sph(x_hbm, out_hbm, in0, in1, ov0, ov1, si0, si1, so0, so1):
        wid = lax.axis_index("s") * mesh.num_cores + lax.axis_index("c")
        b = wid // w_per_b
        e0 = (wid % w_per_b) * span
        ins = [in0, in1]
        ovs = [ov0, ov1]
        sis = [si0, si1]
        sos = [so0, so1]

        y0 = jnp.full((_L,), _C00, jnp.float32)

        @plsc.parallel_loop(0, chunk, step=_L, unroll=4)
        def fill0(s):
            ov0[pl.ds(s, _L)] = y0
            ov1[pl.ds(s, _L)] = y0

        def start_in(j):
            e = e0 + j * chunk
            p = j % 2
            return [
                pltpu.async_copy(
                    x_hbm.at[pl.ds((b * 3 + k) * plane + e, chunk)],
                    ins[p].at[pl.ds(k * chunk, chunk)],
                    sis[p],
                )
                for k in range(3)
            ]

        def start_out(j):
            e = e0 + j * chunk
            p = j % 2
            return [
                pltpu.async_copy(
                    ovs[p].at[pl.ds(k * chunk, chunk)],
                    out_hbm.at[pl.ds((b * 9 + k) * plane + e, chunk)],
                    sos[p],
                )
                for k in range(9)
            ]

        def compute(p):
            iv = ins[p]
            ov = ovs[p]

            @plsc.parallel_loop(0, chunk, step=_L, unroll=4)
            def vec(s):
                x = iv[pl.ds(s, _L)]
                y = iv[pl.ds(chunk + s, _L)]
                z = iv[pl.ds(2 * chunk + s, _L)]
                r2 = x * x + y * y + z * z + 1e-12
                rin = _rsqrt(r2)
                xn = x * rin
                yn = y * rin
                zn = z * rin
                ov[pl.ds(chunk + s, _L)] = _C1 * yn
                ov[pl.ds(2 * chunk + s, _L)] = _C1 * zn
                ov[pl.ds(3 * chunk + s, _L)] = _C1 * xn
                ov[pl.ds(4 * chunk + s, _L)] = _C2A * (xn * yn)
                ov[pl.ds(5 * chunk + s, _L)] = _C2A * (yn * zn)
                ov[pl.ds(6 * chunk + s, _L)] = _C20 * (3.0 * (zn * zn) - 1.0)
                ov[pl.ds(7 * chunk + s, _L)] = _C2A * (xn * zn)
                ov[pl.ds(8 * chunk + s, _L)] = _C22 * (xn * xn - yn * yn)

        hin = {0: start_in(0)}
        hout = {}
        for j in range(nch):
            p = j % 2
            for h in hin.pop(j):
                h.wait()
            if j + 1 < nch:
                hin[j + 1] = start_in(j + 1)
            if j >= 2:
                for h in hout.pop(j - 2):
                    h.wait()
            compute(p)
            hout[j] = start_out(j)
        for j in sorted(hout):
            for h in hout.pop(j):
                h.wait()

    return sph


def kernel(X):
    b, n1, n2, _ = X.shape
    plane = n1 * n2
    xp = jnp.transpose(X, (0, 3, 1, 2))
    xp = xp.reshape(b, 3, n1 // 8, 8, n2 // 128, 128)
    xp = jnp.transpose(xp, (0, 1, 2, 4, 3, 5))
    xf = xp.reshape(b * plane * 3)
    of = _make_sph(b, plane)(xf)
    o = of.reshape(b, 9, n1 // 8, n2 // 128, 8, 128)
    o = jnp.transpose(o, (0, 2, 4, 3, 5, 1))
    return o.reshape(b, n1, n2, 9)
